# baseline (speedup 1.0000x reference)
import functools
import math

import jax
import jax.numpy as jnp
from jax import lax
from jax.experimental import pallas as pl
from jax.experimental.pallas import tpu as pltpu
from jax.experimental.pallas import tpu_sc as plsc

D_MODEL = 64
SCALE = math.sqrt(D_MODEL)

NC = 2
NS = 16
NW = NC * NS
LANES = 16

TBLK = 128
G = 4
S = 4
X = 8
LG = 2
LX = 4


@functools.lru_cache(maxsize=None)
def _build(seq: int, n_tok: int, vpairs: int, d: int):
    n_tb = n_tok // TBLK
    tb_per_w = n_tb // NW
    n_chunks = seq * tb_per_w
    ngrp = TBLK // LANES
    mesh = plsc.VectorSubcoreMesh(core_axis_name="c", subcore_axis_name="s")

    @functools.partial(
        pl.kernel,
        out_type=jax.ShapeDtypeStruct((seq, d, n_tok), jnp.float32),
        mesh=mesh,
        scratch_types=[
            pltpu.VMEM((X, TBLK), jnp.int32),
            pltpu.VMEM((G, TBLK), jnp.int32),
            pltpu.VMEM((G, TBLK), jnp.int32),
            pltpu.VMEM((G, TBLK, 2 * d), jnp.float32),
            pltpu.VMEM((S, d, TBLK), jnp.float32),
        ]
        + [pltpu.SemaphoreType.DMA] * (G + S + X),
        compiler_params=pltpu.CompilerParams(
            use_tc_tiling_on_sc=True,
            needs_layout_passes=False,
            disable_bounds_checks=True,
        ),
    )
    def emb_kernel(xt_hbm, tab_hbm, out_hbm, xbuf, gidx, half, gbuf, obuf, *sems):
        gsem = sems[:G]
        wsem = sems[G : G + S]
        xsem = sems[G + S :]
        wid = lax.axis_index("s") * NC + lax.axis_index("c")
        tb0 = wid * tb_per_w

        def chunk_pos(n):
            return n // tb_per_w, tb0 + lax.rem(n, tb_per_w)

        def issue_xfetch(n, xs):
            p, tb = chunk_pos(n)
            pltpu.async_copy(
                xt_hbm.at[p, pl.ds(tb * TBLK, TBLK)], xbuf.at[xs], xsem[xs]
            )

        def wait_xfetch(n, xs):
            p, tb = chunk_pos(n)
            pltpu.make_async_copy(
                xt_hbm.at[p, pl.ds(tb * TBLK, TBLK)], xbuf.at[xs], xsem[xs]
            ).wait()

        def prep_and_gather(n, gs, xs):
            wait_xfetch(n, xs)
            for g in range(ngrp):
                sl = pl.ds(g * LANES, LANES)
                xv = xbuf[xs, sl]
                gidx[gs, sl] = jnp.bitwise_or(
                    lax.shift_left(lax.shift_right_logical(xv, 15), 14),
                    jnp.bitwise_and(xv, 16383),
                )
                half[gs, sl] = lax.shift_left(
                    jnp.bitwise_and(lax.shift_right_logical(xv, 14), 1), 6
                )
            pltpu.async_copy(tab_hbm.at[gidx.at[gs]], gbuf.at[gs], gsem[gs])

        def wait_gather(gs):
            pltpu.make_async_copy(
                tab_hbm.at[gidx.at[gs]], gbuf.at[gs], gsem[gs]
            ).wait()

        def consume(gs, ss):
            iot = lax.iota(jnp.int32, LANES)
            tokvs = [iot + (g * LANES) for g in range(ngrp)]
            colbs = [half[gs, pl.ds(g * LANES, LANES)] for g in range(ngrp)]
            g2 = gbuf.at[gs]
            o2 = obuf.at[ss]

            def dd_body(j, carry):
                land = jnp.bitwise_and(iot + j, d - 1)
                for g in range(ngrp):
                    v = plsc.load_gather(g2, [tokvs[g], colbs[g] + land])
                    plsc.store_scatter(o2, [land, tokvs[g]], v)
                return carry

            lax.fori_loop(0, d, dd_body, 0, unroll=8)

        def issue_write(n, ss):
            p, tb = chunk_pos(n)
            pltpu.async_copy(
                obuf.at[ss], out_hbm.at[p, :, pl.ds(tb * TBLK, TBLK)], wsem[ss]
            )

        def wait_write(n, ss):
            p, tb = chunk_pos(n)
            pltpu.make_async_copy(
                obuf.at[ss], out_hbm.at[p, :, pl.ds(tb * TBLK, TBLK)], wsem[ss]
            ).wait()

        for q in range(LX):
            issue_xfetch(q, q)
        for m in range(LG):
            prep_and_gather(m, m, m)

        def outer(o, carry):
            for b in range(2 * G):
                n = o * (2 * G) + b
                gs = b % G
                ss = b % S
                wait_gather(gs)

                @pl.when(n >= S)
                def _():
                    wait_write(n - S, ss)

                consume(gs, ss)
                issue_write(n, ss)

                m = n + LG

                @pl.when(m < n_chunks)
                def _():
                    prep_and_gather(m, (b + LG) % G, (b + LG) % X)

                q = n + LX

                @pl.when(q < n_chunks)
                def _():
                    issue_xfetch(q, (b + LX) % X)

            return carry

        lax.fori_loop(0, n_chunks // (2 * G), outer, 0)

        for k in range(S):
            wait_write(n_chunks - S + k, k)

    return emb_kernel


_SB = 32768
_HB = _SB // 2


@functools.lru_cache(maxsize=None)
def _fold_tc(vocab: int, d: int):
    grid = (vocab + _SB - 1) // _SB

    def body(in_ref, out_ref):
        x = in_ref[...] * SCALE
        out_ref[:, 0:d] = x[:, 0:_HB].T
        out_ref[:, d : 2 * d] = x[:, _HB:_SB].T

    return pl.pallas_call(
        body,
        grid=(grid,),
        in_specs=[pl.BlockSpec((d, _SB), lambda i: (0, i))],
        out_specs=pl.BlockSpec((_HB, 2 * d), lambda i: (i, 0)),
        out_shape=jax.ShapeDtypeStruct((grid * _HB, 2 * d), jnp.float32),
    )


def kernel(x, table):
    n_tok, seq = x.shape
    vocab, d = table.shape
    xt = x.T.astype(jnp.int32)
    t2 = _fold_tc(vocab, d)(table.T)
    out = _build(seq, n_tok, vocab // 2, d)(xt, t2)
    return jnp.transpose(out, (2, 0, 1))


# speedup vs baseline: 1.0745x; 1.0745x over previous
import functools
import math

import jax
import jax.numpy as jnp
from jax import lax
from jax.experimental import pallas as pl
from jax.experimental.pallas import tpu as pltpu
from jax.experimental.pallas import tpu_sc as plsc

D_MODEL = 64
SCALE = math.sqrt(D_MODEL)

NC = 2
NS = 16
NW = NC * NS
LANES = 16

TBLK = 128
G = 4
S = 4
X = 8
LG = 3
LX = 4


@functools.lru_cache(maxsize=None)
def _build(seq: int, n_tok: int, vpairs: int, d: int):
    n_tb = n_tok // TBLK
    tb_per_w = n_tb // NW
    n_chunks = seq * tb_per_w
    ngrp = TBLK // LANES
    mesh = plsc.VectorSubcoreMesh(core_axis_name="c", subcore_axis_name="s")

    @functools.partial(
        pl.kernel,
        out_type=jax.ShapeDtypeStruct((seq, d, n_tok), jnp.float32),
        mesh=mesh,
        scratch_types=[
            pltpu.VMEM((X, TBLK), jnp.int32),
            pltpu.VMEM((G, TBLK), jnp.int32),
            pltpu.VMEM((G, TBLK), jnp.int32),
            pltpu.VMEM((G, TBLK, 2 * d), jnp.float32),
            pltpu.VMEM((S, d, TBLK), jnp.float32),
        ]
        + [pltpu.SemaphoreType.DMA] * (G + S + X),
        compiler_params=pltpu.CompilerParams(
            use_tc_tiling_on_sc=True,
            needs_layout_passes=False,
            disable_bounds_checks=True,
        ),
    )
    def emb_kernel(xt_hbm, tab_hbm, out_hbm, xbuf, gidx, half, gbuf, obuf, *sems):
        gsem = sems[:G]
        wsem = sems[G : G + S]
        xsem = sems[G + S :]
        wid = lax.axis_index("s") * NC + lax.axis_index("c")
        tb0 = wid * tb_per_w

        def chunk_pos(n):
            return n // tb_per_w, tb0 + lax.rem(n, tb_per_w)

        def issue_xfetch(n, xs):
            p, tb = chunk_pos(n)
            pltpu.async_copy(
                xt_hbm.at[p, pl.ds(tb * TBLK, TBLK)], xbuf.at[xs], xsem[xs]
            )

        def wait_xfetch(n, xs):
            p, tb = chunk_pos(n)
            pltpu.make_async_copy(
                xt_hbm.at[p, pl.ds(tb * TBLK, TBLK)], xbuf.at[xs], xsem[xs]
            ).wait()

        def prep_and_gather(n, gs, xs):
            wait_xfetch(n, xs)
            for g in range(ngrp):
                sl = pl.ds(g * LANES, LANES)
                xv = xbuf[xs, sl]
                gidx[gs, sl] = jnp.bitwise_or(
                    lax.shift_left(lax.shift_right_logical(xv, 15), 14),
                    jnp.bitwise_and(xv, 16383),
                )
                half[gs, sl] = lax.shift_left(
                    jnp.bitwise_and(lax.shift_right_logical(xv, 14), 1), 6
                )
            pltpu.async_copy(tab_hbm.at[gidx.at[gs]], gbuf.at[gs], gsem[gs])

        def wait_gather(gs):
            pltpu.make_async_copy(
                tab_hbm.at[gidx.at[gs]], gbuf.at[gs], gsem[gs]
            ).wait()

        def consume(gs, ss):
            iot = lax.iota(jnp.int32, LANES)
            tokvs = [iot + (g * LANES) for g in range(ngrp)]
            colbs = [half[gs, pl.ds(g * LANES, LANES)] for g in range(ngrp)]
            g2 = gbuf.at[gs]
            o2 = obuf.at[ss]

            def dd_body(j, carry):
                land = jnp.bitwise_and(iot + j, d - 1)
                for g in range(ngrp):
                    v = plsc.load_gather(g2, [tokvs[g], colbs[g] + land])
                    plsc.store_scatter(o2, [land, tokvs[g]], v)
                return carry

            lax.fori_loop(0, d, dd_body, 0, unroll=4)

        def issue_write(n, ss):
            p, tb = chunk_pos(n)
            pltpu.async_copy(
                obuf.at[ss], out_hbm.at[p, :, pl.ds(tb * TBLK, TBLK)], wsem[ss]
            )

        def wait_write(n, ss):
            p, tb = chunk_pos(n)
            pltpu.make_async_copy(
                obuf.at[ss], out_hbm.at[p, :, pl.ds(tb * TBLK, TBLK)], wsem[ss]
            ).wait()

        for q in range(LX):
            issue_xfetch(q, q)
        for m in range(LG):
            prep_and_gather(m, m, m)

        def outer(o, carry):
            for b in range(2 * G):
                n = o * (2 * G) + b
                gs = b % G
                ss = b % S
                wait_gather(gs)

                @pl.when(n >= S)
                def _():
                    wait_write(n - S, ss)

                consume(gs, ss)
                issue_write(n, ss)

                m = n + LG

                @pl.when(m < n_chunks)
                def _():
                    prep_and_gather(m, (b + LG) % G, (b + LG) % X)

                q = n + LX

                @pl.when(q < n_chunks)
                def _():
                    issue_xfetch(q, (b + LX) % X)

            return carry

        lax.fori_loop(0, n_chunks // (2 * G), outer, 0)

        for k in range(S):
            wait_write(n_chunks - S + k, k)

    return emb_kernel


_SB = 32768
_HB = _SB // 2


@functools.lru_cache(maxsize=None)
def _fold_tc(vocab: int, d: int):
    grid = (vocab + _SB - 1) // _SB

    def body(in_ref, out_ref):
        x = in_ref[...] * SCALE
        out_ref[:, 0:d] = x[:, 0:_HB].T
        out_ref[:, d : 2 * d] = x[:, _HB:_SB].T

    return pl.pallas_call(
        body,
        grid=(grid,),
        in_specs=[pl.BlockSpec((d, _SB), lambda i: (0, i))],
        out_specs=pl.BlockSpec((_HB, 2 * d), lambda i: (i, 0)),
        out_shape=jax.ShapeDtypeStruct((grid * _HB, 2 * d), jnp.float32),
    )


def kernel(x, table):
    n_tok, seq = x.shape
    vocab, d = table.shape
    xt = x.T.astype(jnp.int32)
    t2 = _fold_tc(vocab, d)(table.T)
    out = _build(seq, n_tok, vocab // 2, d)(xt, t2)
    return jnp.transpose(out, (2, 0, 1))
